# baseline (speedup 1.0000x reference)
import functools

import jax
import jax.numpy as jnp
from jax import lax
from jax.experimental import pallas as pl
from jax.experimental.pallas import tpu as pltpu
from jax.experimental.pallas import tpu_sc as plsc

B, L, N = 16, 50, 1000
ALPHA = 0.7
P = L - 1
PPAD = 64
CHUNK = 128


def _dense_tc_kernel(x_ref, tgt_ref, gid_ref, seq_ref):
    x = x_ref[...]
    tgt = tgt_ref[...]

    m = jnp.max(x, axis=-1, keepdims=True)
    lane = lax.broadcasted_iota(jnp.int32, (B, L, N), 2)

    idx = jnp.min(jnp.where(x == m, lane, N), axis=-1)

    lse = m[..., 0] + jnp.log(jnp.sum(jnp.exp(x - m), axis=-1))
    x_t = jnp.sum(jnp.where(lane == tgt[..., None], x, 0.0), axis=-1)
    seq = jnp.mean(lse - x_t)

    src = idx[:, :P]
    dst = idx[:, 1:]
    b_iota = lax.broadcasted_iota(jnp.int32, (B, P), 0)
    g = b_iota * (N * N) + src * N + dst

    zpad = jnp.zeros((B, PPAD - P), jnp.int32)
    gid_ref[...] = jnp.concatenate([g, zpad], axis=-1)
    seq_ref[...] = jnp.full((1, 16), seq, jnp.float32)


def _sc_body(adj_ref, gid_ref, seq_ref, out_ref,
             gid_v, vals_v, seq_v, out_v, sem):
    wid = lax.axis_index("s") * 2 + lax.axis_index("c")

    @pl.when(wid == 0)
    def _():
        pltpu.sync_copy(gid_ref, gid_v)
        pltpu.sync_copy(seq_ref, seq_v)

        copies = []
        for k in range(B * PPAD // CHUNK):
            sl = pl.ds(k * CHUNK, CHUNK)
            copies.append(pltpu.async_copy(adj_ref.at[gid_v.at[sl]],
                                           vals_v.at[sl], sem))
        for c in copies:
            c.wait()

        tl = lax.iota(jnp.int32, 16)
        bal = jnp.float32(0.0)
        for b in range(B):
            acc = jnp.zeros((16,), jnp.float32)
            for c in range(PPAD // 16):
                vals = vals_v[pl.ds(b * PPAD + c * 16, 16)]
                acc = acc + jnp.where(tl + c * 16 < P, vals, 0.0)
            s = acc[0]
            for i in range(1, 16):
                s = s + acc[i]
            bal = bal + jnp.abs(s)
        seq = seq_v[...][0]
        final = ALPHA * (bal * (1.0 / B)) + (1.0 - ALPHA) * seq
        out_v[...] = jnp.full((16,), final, jnp.float32)
        pltpu.sync_copy(out_v, out_ref)


def kernel(path_logits, target_paths, adj_matrix):
    gid, seq = pl.pallas_call(
        _dense_tc_kernel,
        out_shape=[
            jax.ShapeDtypeStruct((B, PPAD), jnp.int32),
            jax.ShapeDtypeStruct((1, 16), jnp.float32),
        ],
    )(path_logits, target_paths.astype(jnp.int32))

    mesh = plsc.VectorSubcoreMesh(core_axis_name="c", subcore_axis_name="s")
    sc = functools.partial(
        pl.kernel,
        mesh=mesh,
        out_type=jax.ShapeDtypeStruct((16,), jnp.float32),
        scratch_types=[
            pltpu.VMEM((B * PPAD,), jnp.int32),
            pltpu.VMEM((B * PPAD,), jnp.float32),
            pltpu.VMEM((16,), jnp.float32),
            pltpu.VMEM((16,), jnp.float32),
            pltpu.SemaphoreType.DMA,
        ],
    )(_sc_body)
    out = sc(adj_matrix.reshape(B * N * N),
             gid.reshape(B * PPAD),
             seq.reshape(16))
    return out[0]
